# baseline (speedup 1.0000x reference)
import functools

import jax
import jax.numpy as jnp
from jax import lax
from jax.experimental import pallas as pl
from jax.experimental.pallas import tpu as pltpu
from jax.experimental.pallas import tpu_sc as plsc

N_V = 6890
P = 8192
NC = 2
NS = 16
NW = NC * NS
R = P // NW
L = 16
G = R // L
SCALE = 1.0 / (N_V * 3)

_mesh = plsc.VectorSubcoreMesh(core_axis_name="c", subcore_axis_name="s")


@functools.partial(
    pl.kernel,
    mesh=_mesh,
    compiler_params=pltpu.CompilerParams(needs_layout_passes=False),
    out_type=jax.ShapeDtypeStruct((NC, L), jnp.float32),
    scratch_types=[
        pltpu.VMEM((P * 3,), jnp.float32),
        pltpu.VMEM((R,), jnp.int32),
        pltpu.VMEM((R,), jnp.float32),
        pltpu.VMEM((L,), jnp.float32),
        pltpu.VMEM((NS, L), jnp.float32),
        pltpu.HBM((NC, NS, L), jnp.float32),
    ],
)
def _sym_loss_kernel(v_hbm, w_hbm, idx_hbm, out_hbm,
                     vfull, idx_v, w_v, stage_v, acc_v, partials):
    c = lax.axis_index("c")
    s = lax.axis_index("s")
    wid = c * NS + s
    base = wid * R

    pltpu.sync_copy(v_hbm, vfull)
    pltpu.sync_copy(idx_hbm.at[pl.ds(base, R)], idx_v)
    pltpu.sync_copy(w_hbm.at[pl.ds(base, R)], w_v)

    lanes3 = lax.iota(jnp.int32, L) * 3
    own0 = lanes3 + (base * 3)

    acc = jnp.zeros((L,), jnp.float32)
    for t in range(G):
        idx16 = idx_v[pl.ds(t * L, L)]
        fi = idx16 * 3
        ri = own0 + (t * L * 3)
        vx = plsc.load_gather(vfull, [ri])
        vy = plsc.load_gather(vfull, [ri + 1])
        vz = plsc.load_gather(vfull, [ri + 2])
        mx = plsc.load_gather(vfull, [fi])
        my = plsc.load_gather(vfull, [fi + 1])
        mz = plsc.load_gather(vfull, [fi + 2])
        w16 = w_v[pl.ds(t * L, L)]
        dx = vx + mx
        dy = vy - my
        dz = vz - mz
        acc = acc + w16 * (dx * dx + dy * dy + dz * dz)

    stage_v[...] = acc
    pltpu.sync_copy(stage_v, partials.at[c].at[s])
    plsc.subcore_barrier()

    @pl.when(s == 0)
    def _leader():
        pltpu.sync_copy(partials.at[c], acc_v)
        vec = acc_v[0]
        for i in range(1, NS):
            vec = vec + acc_v[i]
        total = jnp.sum(vec) * SCALE
        stage_v[...] = jnp.full((L,), total, jnp.float32)
        pltpu.sync_copy(stage_v, out_hbm.at[c])


def kernel(v, symmetry_w, idx):
    v_p = jnp.zeros((P * 3,), jnp.float32).at[: N_V * 3].set(v.reshape(-1))
    w_p = jnp.zeros((P,), jnp.float32).at[:N_V].set(symmetry_w.reshape(-1))
    idx_p = jnp.zeros((P,), jnp.int32).at[:N_V].set(idx.astype(jnp.int32))
    out = _sym_loss_kernel(v_p, w_p, idx_p)
    return out[0, 0] + out[1, 0]


# speedup vs baseline: 1.2984x; 1.2984x over previous
import functools

import jax
import jax.numpy as jnp
from jax import lax
from jax.experimental import pallas as pl
from jax.experimental.pallas import tpu as pltpu
from jax.experimental.pallas import tpu_sc as plsc

N_V = 6890
NS = 16
L = 16
R = 432
G = R // L
TAIL_W = NS - 1
TAIL_VALID = N_V - TAIL_W * R
TAIL_G = (TAIL_VALID + L - 1) // L
TAIL_REM = TAIL_VALID - (TAIL_G - 1) * L
NW3 = N_V * 3
SCALE = 1.0 / NW3

_mesh = plsc.VectorSubcoreMesh(
    core_axis_name="c", subcore_axis_name="s", num_cores=1
)


@functools.partial(
    pl.kernel,
    mesh=_mesh,
    compiler_params=pltpu.CompilerParams(needs_layout_passes=False),
    out_type=jax.ShapeDtypeStruct((1,), jnp.float32),
    scratch_types=[
        pltpu.VMEM((NW3,), jnp.float32),
        pltpu.VMEM((R,), jnp.int32),
        pltpu.VMEM((R,), jnp.float32),
        pltpu.VMEM((L,), jnp.float32),
        pltpu.VMEM((NS, L), jnp.float32),
        pltpu.HBM((NS, L), jnp.float32),
        pltpu.SemaphoreType.DMA,
    ],
)
def _sym_loss_kernel(v_hbm, w_hbm, idx_hbm, out_hbm,
                     vfull, idx_v, w_v, stage_v, acc_v, partials, sem):
    s = lax.axis_index("s")
    base = s * R

    lanes = lax.iota(jnp.int32, L)
    zeros_i = jnp.zeros((L,), jnp.int32)
    zeros_f = jnp.zeros((L,), jnp.float32)

    def stream_table():
        copies = []
        for off, ln in ((0, 5168), (5168, 5168), (10336, 5168), (15504, 5166)):
            copies.append(pltpu.async_copy(
                v_hbm.at[pl.ds(off, ln)], vfull.at[pl.ds(off, ln)], sem))
        return copies

    def group_contrib(t, rows):
        idx16 = idx_v[pl.ds(t * L, L)]
        fi = idx16 * 3
        ri = rows * 3
        vx = plsc.load_gather(vfull, [ri])
        vy = plsc.load_gather(vfull, [ri + 1])
        vz = plsc.load_gather(vfull, [ri + 2])
        mx = plsc.load_gather(vfull, [fi])
        my = plsc.load_gather(vfull, [fi + 1])
        mz = plsc.load_gather(vfull, [fi + 2])
        w16 = w_v[pl.ds(t * L, L)]
        dx = vx + mx
        dy = vy - my
        dz = vz - mz
        return w16 * (dx * dx + dy * dy + dz * dz)

    @pl.when(s < TAIL_W)
    def _full_tile():
        copies = stream_table()
        pltpu.sync_copy(idx_hbm.at[pl.ds(base, R)], idx_v)
        pltpu.sync_copy(w_hbm.at[pl.ds(base, R)], w_v)
        for cp in copies:
            cp.wait()
        acc = jnp.zeros((L,), jnp.float32)
        for t in range(G):
            acc = acc + group_contrib(t, lanes + base + t * L)
        stage_v[...] = acc

    @pl.when(s == TAIL_W)
    def _tail_tile():
        copies = stream_table()
        pre = (TAIL_VALID // L) * L
        idx_v[pl.ds(pre, L)] = zeros_i
        idx_v[pl.ds(pre + L, L)] = zeros_i
        w_v[pl.ds(pre, L)] = zeros_f
        w_v[pl.ds(pre + L, L)] = zeros_f
        tbase = TAIL_W * R
        pltpu.sync_copy(idx_hbm.at[pl.ds(tbase, TAIL_VALID)],
                        idx_v.at[pl.ds(0, TAIL_VALID)])
        pltpu.sync_copy(w_hbm.at[pl.ds(tbase, TAIL_VALID)],
                        w_v.at[pl.ds(0, TAIL_VALID)])
        for cp in copies:
            cp.wait()
        acc = jnp.zeros((L,), jnp.float32)
        for t in range(TAIL_G - 1):
            acc = acc + group_contrib(t, lanes + tbase + t * L)
        last = TAIL_G - 1
        rows = jnp.minimum(lanes + tbase + last * L, N_V - 1)
        contrib = group_contrib(last, rows)
        acc = acc + jnp.where(lanes < TAIL_REM, contrib, zeros_f)
        stage_v[...] = acc

    pltpu.sync_copy(stage_v, partials.at[s])
    plsc.subcore_barrier()

    @pl.when(s == 0)
    def _leader():
        pltpu.sync_copy(partials, acc_v)
        vec = acc_v[0]
        for i in range(1, NS):
            vec = vec + acc_v[i]
        total = jnp.sum(vec) * SCALE
        stage_v[...] = jnp.full((L,), total, jnp.float32)
        pltpu.sync_copy(stage_v.at[pl.ds(0, 1)], out_hbm)


def kernel(v, symmetry_w, idx):
    out = _sym_loss_kernel(v.reshape(-1), symmetry_w.reshape(-1),
                           idx.astype(jnp.int32))
    return out.reshape(())
